# baseline (speedup 1.0000x reference)
import functools

import jax
import jax.numpy as jnp
from jax import lax
from jax.experimental import pallas as pl
from jax.experimental.pallas import tpu as pltpu
from jax.experimental.pallas import tpu_sc as plsc

N_NODES = 10000
N_USERS = 4000
NPAD = 10240
DIM = 256
HEADS1, OUT1 = 4, 256
HEADS2, OUT2 = 1, 256
B = 4096

NC, NS, LANES = 2, 16, 16
NW = NC * NS
KB = 128
KA = 256
TSL = NPAD // NS


def _mesh():
    return plsc.VectorSubcoreMesh(core_axis_name="c", subcore_axis_name="s",
                                  num_cores=NC, num_subcores=NS)


def _wid():
    return lax.axis_index("s") * NC + lax.axis_index("c")


def _tc1(embp, W1, b1r, AmatT):
    BN = 1024

    def body(x_ref, w_ref, b_ref, at_ref, h_ref, aux_ref):
        h = jnp.dot(x_ref[...], w_ref[...], preferred_element_type=jnp.float32)
        aux_ref[...] = lax.dot_general(
            at_ref[...], h, (((1,), (1,)), ((), ())),
            preferred_element_type=jnp.float32)
        hb = h + b_ref[...]
        for c in range(8):
            h_ref[c] = hb[:, c * 128:(c + 1) * 128]

    return pl.pallas_call(
        body,
        grid=(NPAD // BN,),
        in_specs=[pl.BlockSpec((BN, DIM), lambda i: (i, 0)),
                  pl.BlockSpec((DIM, 1024), lambda i: (0, 0)),
                  pl.BlockSpec((1, 1024), lambda i: (0, 0)),
                  pl.BlockSpec((8, 1024), lambda i: (0, 0))],
        out_specs=[pl.BlockSpec((8, BN, 128), lambda i: (0, i, 0)),
                   pl.BlockSpec((8, BN), lambda i: (0, i))],
        out_shape=[jax.ShapeDtypeStruct((8, NPAD, 128), jnp.float32),
                   jax.ShapeDtypeStruct((8, NPAD), jnp.float32)],
    )(embp, W1, b1r, AmatT)


def _tc2(x1c, W2, b2r, AmatT2):
    BN = 1024

    def body(x_ref, w_ref, b_ref, at_ref, h_ref, aux_ref):
        acc = jnp.zeros((BN, 256), jnp.float32)
        for c in range(8):
            xc = x_ref[c]
            xc = jnp.where(xc > 0, xc, jnp.exp(xc) - 1.0)
            acc = acc + jnp.dot(xc, w_ref[pl.ds(c * 128, 128), :],
                                preferred_element_type=jnp.float32)
        aux_ref[...] = lax.dot_general(
            at_ref[...], acc, (((1,), (1,)), ((), ())),
            preferred_element_type=jnp.float32)
        hb = acc + b_ref[...]
        for c in range(2):
            h_ref[c] = hb[:, c * 128:(c + 1) * 128]

    return pl.pallas_call(
        body,
        grid=(NPAD // BN,),
        in_specs=[pl.BlockSpec((8, BN, 128), lambda i: (0, i, 0)),
                  pl.BlockSpec((1024, 256), lambda i: (0, 0)),
                  pl.BlockSpec((1, 256), lambda i: (0, 0)),
                  pl.BlockSpec((8, 256), lambda i: (0, 0))],
        out_specs=[pl.BlockSpec((2, BN, 128), lambda i: (0, i, 0)),
                   pl.BlockSpec((8, BN), lambda i: (0, i))],
        out_shape=[jax.ShapeDtypeStruct((2, NPAD, 128), jnp.float32),
                   jax.ShapeDtypeStruct((8, NPAD), jnp.float32)],
    )(x1c, W2, b2r, AmatT2)


def _sc_edge_pass(nh, epad, srcp, dstp, auxT):
    ept = epad // NW
    nblk = ept // KA

    @functools.partial(
        pl.kernel,
        out_type=[jax.ShapeDtypeStruct((nh * epad,), jnp.float32),
                  jax.ShapeDtypeStruct((NW * nh * NPAD,), jnp.float32)],
        mesh=_mesh(),
        compiler_params=pltpu.CompilerParams(needs_layout_passes=False),
        scratch_types=[
            pltpu.VMEM((2 * nh * NPAD,), jnp.float32),
            pltpu.VMEM((nh * NPAD,), jnp.float32),
            pltpu.VMEM((KA,), jnp.int32),
            pltpu.VMEM((KA,), jnp.int32),
            pltpu.VMEM((nh * KA,), jnp.float32),
        ])
    def k(src_hbm, dst_hbm, aux_hbm, ee_hbm, denp_hbm,
          aux_t, den_t, src_t, dst_t, ee_t):
        wid = _wid()
        pltpu.sync_copy(aux_hbm.at[pl.ds(0, 2 * nh * NPAD)], aux_t)
        zv = jnp.zeros((LANES,), jnp.float32)

        def zb(i, carry):
            den_t[pl.ds(i * LANES, LANES)] = zv
            return carry
        lax.fori_loop(0, nh * NPAD // LANES, zb, 0)
        base = pl.multiple_of(wid * ept, KA)

        def blk(bi, carry):
            off = pl.multiple_of(base + bi * KA, KA)
            pltpu.sync_copy(src_hbm.at[pl.ds(off, KA)], src_t)
            pltpu.sync_copy(dst_hbm.at[pl.ds(off, KA)], dst_t)

            def vec(i, carry2):
                sl = pl.ds(i * LANES, LANES)
                sv = src_t[sl]
                dv = dst_t[sl]
                for h in range(nh):
                    asv = plsc.load_gather(aux_t, [sv + (h * NPAD)])
                    adv = plsc.load_gather(aux_t, [dv + ((nh + h) * NPAD)])
                    e = asv + adv
                    e = jnp.maximum(e, 0.2 * e)
                    eev = jnp.exp(e)
                    ee_t[pl.ds(h * KA + i * LANES, LANES)] = eev
                    plsc.addupdate_scatter(den_t, [dv + (h * NPAD)], eev)
                return carry2
            lax.fori_loop(0, KA // LANES, vec, 0)
            for h in range(nh):
                pltpu.sync_copy(ee_t.at[pl.ds(h * KA, KA)],
                                ee_hbm.at[pl.ds(h * epad + off, KA)])
            return carry
        lax.fori_loop(0, nblk, blk, 0)
        pltpu.sync_copy(den_t,
                        denp_hbm.at[pl.ds(wid * (nh * NPAD), nh * NPAD)])

    return k(srcp, dstp, auxT)


def _sc_den_reduce(nh, denp):
    SL = NPAD // NW

    @functools.partial(
        pl.kernel,
        out_type=jax.ShapeDtypeStruct((nh * NPAD,), jnp.float32),
        mesh=_mesh(),
        compiler_params=pltpu.CompilerParams(needs_layout_passes=False),
        scratch_types=[pltpu.VMEM((nh * SL,), jnp.float32),
                       pltpu.VMEM((SL,), jnp.float32)])
    def k(denp_hbm, den_hbm, acc_t, buf_t):
        wid = _wid()
        nbase = pl.multiple_of(wid * SL, SL)
        zv = jnp.zeros((LANES,), jnp.float32)

        def zb(i, carry):
            acc_t[pl.ds(i * LANES, LANES)] = zv
            return carry
        lax.fori_loop(0, nh * SL // LANES, zb, 0)

        def tb(t, carry):
            for h in range(nh):
                pltpu.sync_copy(
                    denp_hbm.at[pl.ds(t * (nh * NPAD) + h * NPAD + nbase, SL)],
                    buf_t)

                def ab(i, carry2, h=h):
                    sl = pl.ds(h * SL + i * LANES, LANES)
                    slb = pl.ds(i * LANES, LANES)
                    acc_t[sl] = acc_t[sl] + buf_t[slb]
                    return carry2
                lax.fori_loop(0, SL // LANES, ab, 0)
            return carry
        lax.fori_loop(0, NW, tb, 0)
        for h in range(nh):
            pltpu.sync_copy(acc_t.at[pl.ds(h * SL, SL)],
                            den_hbm.at[pl.ds(h * NPAD + nbase, SL)])

    return k(denp)


def _sc_spmm(nh, nchunks, epad, srcp, dstp, ee, den, hc):
    cps = nchunks // NC
    ept = epad // NS
    nblk = ept // KB

    @functools.partial(
        pl.kernel,
        out_type=jax.ShapeDtypeStruct((nchunks * NPAD, 128), jnp.float32),
        mesh=_mesh(),
        compiler_params=pltpu.CompilerParams(needs_layout_passes=False),
        scratch_types=[
            pltpu.VMEM_SHARED((NPAD, 128), jnp.float32),
            pltpu.VMEM((KB,), jnp.int32),
            pltpu.VMEM((KB,), jnp.int32),
            pltpu.VMEM((KB,), jnp.int32),
            pltpu.VMEM((KB,), jnp.float32),
            pltpu.VMEM((KB, 128), jnp.float32),
            pltpu.VMEM((NPAD,), jnp.float32),
            pltpu.VMEM((KB, 128), jnp.float32),
            pltpu.VMEM((TSL,), jnp.float32),
            pltpu.SemaphoreType.DMA,
        ])
    def k(src_hbm, dst_hbm, ee_hbm, den_hbm, h_hbm, out_hbm,
          acc_sh, src_t, dst_t, idx_t, ee_t, msg_t, den_t, zbuf_t, inv_t, sem):
        sc = lax.axis_index("c")
        sub = lax.axis_index("s")
        zv = jnp.zeros((LANES,), jnp.float32)

        def zrow(i, carry):
            for r in range(8):
                zbuf_t[i, pl.ds(r * LANES, LANES)] = zv
            return carry
        lax.fori_loop(0, KB, zrow, 0)

        for cj in range(cps):
            c = sc * cps + cj
            head = (c * nh) // nchunks
            cbase = pl.multiple_of(c * NPAD, NPAD)
            pltpu.sync_copy(
                den_hbm.at[pl.ds(pl.multiple_of(head * NPAD, NPAD), NPAD)],
                den_t)
            for j in range(TSL // KB):
                pltpu.sync_copy(
                    zbuf_t, acc_sh.at[pl.ds(sub * TSL + j * KB, KB)])
            plsc.subcore_barrier()

            ebase = pl.multiple_of(sub * ept, KB)

            def blk(bi, carry):
                off = pl.multiple_of(ebase + bi * KB, KB)
                pltpu.sync_copy(src_hbm.at[pl.ds(off, KB)], src_t)
                pltpu.sync_copy(dst_hbm.at[pl.ds(off, KB)], dst_t)
                pltpu.sync_copy(
                    ee_hbm.at[pl.ds(head * epad + off, KB)], ee_t)

                def vi(i, carry2):
                    sl = pl.ds(i * LANES, LANES)
                    idx_t[sl] = src_t[sl] + cbase
                    return carry2
                lax.fori_loop(0, KB // LANES, vi, 0)
                pltpu.async_copy(h_hbm.at[idx_t], msg_t, sem).wait()

                def ei(iv, carry2):
                    av = ee_t[pl.ds(iv * LANES, LANES)]
                    for l in range(LANES):
                        a = av[l]
                        i = iv * LANES + l
                        for r in range(8):
                            sl = pl.ds(r * LANES, LANES)
                            msg_t[i, sl] = msg_t[i, sl] * a
                    return carry2
                lax.fori_loop(0, KB // LANES, ei, 0)
                pltpu.sync_copy(msg_t, acc_sh.at[dst_t], add=True)
                return carry
            lax.fori_loop(0, nblk, blk, 0)
            plsc.subcore_barrier()

            rbase = sub * TSL

            def iv(i, carry):
                sl = pl.ds(i * LANES, LANES)
                inv_t[sl] = 1.0 / (den_t[pl.ds(rbase + i * LANES, LANES)]
                                   + 1e-16)
                return carry
            lax.fori_loop(0, TSL // LANES, iv, 0)

            for j in range(TSL // KB):
                pltpu.sync_copy(acc_sh.at[pl.ds(rbase + j * KB, KB)], msg_t)

                def rr(iv, carry, j=j):
                    sv = inv_t[pl.ds(j * KB + iv * LANES, LANES)]
                    for l in range(LANES):
                        s = sv[l]
                        i = iv * LANES + l
                        for r in range(8):
                            sl = pl.ds(r * LANES, LANES)
                            msg_t[i, sl] = msg_t[i, sl] * s
                    return carry
                lax.fori_loop(0, KB // LANES, rr, 0)
                pltpu.sync_copy(
                    msg_t,
                    out_hbm.at[pl.ds(cbase + rbase + j * KB, KB)])
            plsc.subcore_barrier()

    return k(srcp, dstp, ee, den, hc)


def _sc_lookup(x2, u_idx, i_idx, n_idx):
    PW = B // NW

    out_t = jax.ShapeDtypeStruct((2 * B, 128), jnp.float32)

    @functools.partial(
        pl.kernel,
        out_type=[out_t, out_t, out_t],
        mesh=_mesh(),
        compiler_params=pltpu.CompilerParams(needs_layout_passes=False),
        scratch_types=[
            pltpu.VMEM((PW,), jnp.int32),
            pltpu.VMEM((PW,), jnp.int32),
            pltpu.VMEM((PW, 128), jnp.float32),
            pltpu.SemaphoreType.DMA,
        ])
    def k(x2_hbm, u_hbm, i_hbm, n_hbm, ou_hbm, oi_hbm, on_hbm,
          idx_t, idxc_t, buf_t, sem):
        wid = _wid()
        wbase = pl.multiple_of(wid * PW, PW)
        for a, (src_idx, dst, off) in enumerate(
                [(u_hbm, ou_hbm, 0), (i_hbm, oi_hbm, N_USERS),
                 (n_hbm, on_hbm, N_USERS)]):
            pltpu.sync_copy(src_idx.at[pl.ds(wbase, PW)], idx_t)
            for c in range(2):
                def ai(i, carry, off=off, c=c):
                    sl = pl.ds(i * LANES, LANES)
                    idxc_t[sl] = idx_t[sl] + (off + c * NPAD)
                    return carry
                lax.fori_loop(0, PW // LANES, ai, 0)
                pltpu.async_copy(x2_hbm.at[idxc_t], buf_t, sem).wait()
                pltpu.sync_copy(buf_t, dst.at[pl.ds(c * B + wbase, PW)])

    return k(x2, u_idx, i_idx, n_idx)


def kernel(user_indices, item_indices, bundle_indices, item_indices_negative,
           bundle_indices_negative, edge_index, emb, W1, a_src1, a_dst1, b1,
           W2, a_src2, a_dst2, b2):
    src, dst = edge_index[0], edge_index[1]
    e = src.shape[0]
    gran = NS * 1024
    epad = ((e + gran - 1) // gran) * gran

    srcp = jnp.pad(src.astype(jnp.int32), (0, epad - e))
    dstp = jnp.pad(dst.astype(jnp.int32), (0, epad - e),
                   constant_values=NPAD - 1)
    embp = jnp.pad(emb, ((0, NPAD - emb.shape[0]), (0, 0)))

    eye1 = jnp.eye(HEADS1, dtype=jnp.float32)
    AmatT1 = jnp.concatenate(
        [(eye1[:, :, None] * a_src1[None, :, :]).reshape(HEADS1, HEADS1 * OUT1),
         (eye1[:, :, None] * a_dst1[None, :, :]).reshape(HEADS1, HEADS1 * OUT1)],
        axis=0)
    AmatT2 = jnp.concatenate(
        [a_src2, a_dst2, jnp.zeros((6, OUT2), jnp.float32)], axis=0)

    h1c, aux1 = _tc1(embp, W1, b1.reshape(1, -1), AmatT1)
    ee1, denp1 = _sc_edge_pass(HEADS1, epad, srcp, dstp, aux1.reshape(-1))
    den1 = _sc_den_reduce(HEADS1, denp1)
    x1 = _sc_spmm(HEADS1, 8, epad, srcp, dstp, ee1, den1,
                  h1c.reshape(8 * NPAD, 128))

    h2c, aux2 = _tc2(x1.reshape(8, NPAD, 128), W2, b2.reshape(1, -1), AmatT2)
    ee2, denp2 = _sc_edge_pass(HEADS2, epad, srcp, dstp, aux2.reshape(-1))
    den2 = _sc_den_reduce(HEADS2, denp2)
    x2 = _sc_spmm(HEADS2, 2, epad, srcp, dstp, ee2, den2,
                  h2c.reshape(2 * NPAD, 128))

    ou, oi, on = _sc_lookup(x2, user_indices.astype(jnp.int32),
                            item_indices.astype(jnp.int32),
                            item_indices_negative.astype(jnp.int32))
    user_embeds = jnp.concatenate([ou[:B], ou[B:]], axis=1)
    item_embeds = jnp.concatenate([oi[:B], oi[B:]], axis=1)
    item_embeds_neg = jnp.concatenate([on[:B], on[B:]], axis=1)
    return (user_embeds, item_embeds, item_embeds_neg)


# speedup vs baseline: 1.8673x; 1.8673x over previous
import functools

import jax
import jax.numpy as jnp
from jax import lax
from jax.experimental import pallas as pl
from jax.experimental.pallas import tpu as pltpu
from jax.experimental.pallas import tpu_sc as plsc

N_NODES = 10000
N_USERS = 4000
NPAD = 10240
DIM = 256
HEADS1, OUT1 = 4, 256
HEADS2, OUT2 = 1, 256
B = 4096

NC, NS, LANES = 2, 16, 16
NW = NC * NS
KB = 128
KA = 256
TSL = NPAD // NS


def _mesh():
    return plsc.VectorSubcoreMesh(core_axis_name="c", subcore_axis_name="s",
                                  num_cores=NC, num_subcores=NS)


def _wid():
    return lax.axis_index("s") * NC + lax.axis_index("c")


def _tc1(embp, W1, b1r, AmatT):
    BN = 1024

    def body(x_ref, w_ref, b_ref, at_ref, h_ref, aux_ref):
        h = jnp.dot(x_ref[...], w_ref[...], preferred_element_type=jnp.float32)
        aux_ref[...] = lax.dot_general(
            at_ref[...], h, (((1,), (1,)), ((), ())),
            preferred_element_type=jnp.float32)
        hb = h + b_ref[...]
        for c in range(8):
            h_ref[c] = hb[:, c * 128:(c + 1) * 128]

    return pl.pallas_call(
        body,
        grid=(NPAD // BN,),
        in_specs=[pl.BlockSpec((BN, DIM), lambda i: (i, 0)),
                  pl.BlockSpec((DIM, 1024), lambda i: (0, 0)),
                  pl.BlockSpec((1, 1024), lambda i: (0, 0)),
                  pl.BlockSpec((8, 1024), lambda i: (0, 0))],
        out_specs=[pl.BlockSpec((8, BN, 128), lambda i: (0, i, 0)),
                   pl.BlockSpec((8, BN), lambda i: (0, i))],
        out_shape=[jax.ShapeDtypeStruct((8, NPAD, 128), jnp.float32),
                   jax.ShapeDtypeStruct((8, NPAD), jnp.float32)],
    )(embp, W1, b1r, AmatT)


def _tc2(x1c, W2, b2r, AmatT2):
    BN = 1024

    def body(x_ref, w_ref, b_ref, at_ref, h_ref, aux_ref):
        acc = jnp.zeros((BN, 256), jnp.float32)
        for c in range(8):
            xc = x_ref[c]
            xc = jnp.where(xc > 0, xc, jnp.exp(xc) - 1.0)
            acc = acc + jnp.dot(xc, w_ref[pl.ds(c * 128, 128), :],
                                preferred_element_type=jnp.float32)
        aux_ref[...] = lax.dot_general(
            at_ref[...], acc, (((1,), (1,)), ((), ())),
            preferred_element_type=jnp.float32)
        hb = acc + b_ref[...]
        for c in range(2):
            h_ref[c] = hb[:, c * 128:(c + 1) * 128]

    return pl.pallas_call(
        body,
        grid=(NPAD // BN,),
        in_specs=[pl.BlockSpec((8, BN, 128), lambda i: (0, i, 0)),
                  pl.BlockSpec((1024, 256), lambda i: (0, 0)),
                  pl.BlockSpec((1, 256), lambda i: (0, 0)),
                  pl.BlockSpec((8, 256), lambda i: (0, 0))],
        out_specs=[pl.BlockSpec((2, BN, 128), lambda i: (0, i, 0)),
                   pl.BlockSpec((8, BN), lambda i: (0, i))],
        out_shape=[jax.ShapeDtypeStruct((2, NPAD, 128), jnp.float32),
                   jax.ShapeDtypeStruct((8, NPAD), jnp.float32)],
    )(x1c, W2, b2r, AmatT2)


def _sc_edge_pass(nh, epad, srcp, dstp, auxT):
    ept = epad // NW
    nblk = ept // KA

    @functools.partial(
        pl.kernel,
        out_type=[jax.ShapeDtypeStruct((nh * epad,), jnp.float32),
                  jax.ShapeDtypeStruct((NW * nh * NPAD,), jnp.float32)],
        mesh=_mesh(),
        compiler_params=pltpu.CompilerParams(needs_layout_passes=False),
        scratch_types=[
            pltpu.VMEM((2 * nh * NPAD,), jnp.float32),
            pltpu.VMEM((nh * NPAD,), jnp.float32),
            pltpu.VMEM((KA,), jnp.int32),
            pltpu.VMEM((KA,), jnp.int32),
            pltpu.VMEM((nh * KA,), jnp.float32),
        ])
    def k(src_hbm, dst_hbm, aux_hbm, ee_hbm, denp_hbm,
          aux_t, den_t, src_t, dst_t, ee_t):
        wid = _wid()
        pltpu.sync_copy(aux_hbm.at[pl.ds(0, 2 * nh * NPAD)], aux_t)
        zv = jnp.zeros((LANES,), jnp.float32)

        def zb(i, carry):
            den_t[pl.ds(i * LANES, LANES)] = zv
            return carry
        lax.fori_loop(0, nh * NPAD // LANES, zb, 0)
        base = pl.multiple_of(wid * ept, KA)

        def blk(bi, carry):
            off = pl.multiple_of(base + bi * KA, KA)
            pltpu.sync_copy(src_hbm.at[pl.ds(off, KA)], src_t)
            pltpu.sync_copy(dst_hbm.at[pl.ds(off, KA)], dst_t)

            def vec(i, carry2):
                sl = pl.ds(i * LANES, LANES)
                sv = src_t[sl]
                dv = dst_t[sl]
                for h in range(nh):
                    asv = plsc.load_gather(aux_t, [sv + (h * NPAD)])
                    adv = plsc.load_gather(aux_t, [dv + ((nh + h) * NPAD)])
                    e = asv + adv
                    e = jnp.maximum(e, 0.2 * e)
                    eev = jnp.exp(e)
                    ee_t[pl.ds(h * KA + i * LANES, LANES)] = eev
                    plsc.addupdate_scatter(den_t, [dv + (h * NPAD)], eev)
                return carry2
            lax.fori_loop(0, KA // LANES, vec, 0)
            for h in range(nh):
                pltpu.sync_copy(ee_t.at[pl.ds(h * KA, KA)],
                                ee_hbm.at[pl.ds(h * epad + off, KA)])
            return carry
        lax.fori_loop(0, nblk, blk, 0)
        pltpu.sync_copy(den_t,
                        denp_hbm.at[pl.ds(wid * (nh * NPAD), nh * NPAD)])

    return k(srcp, dstp, auxT)


def _sc_den_reduce(nh, denp):
    SL = NPAD // NW

    @functools.partial(
        pl.kernel,
        out_type=jax.ShapeDtypeStruct((nh * NPAD,), jnp.float32),
        mesh=_mesh(),
        compiler_params=pltpu.CompilerParams(needs_layout_passes=False),
        scratch_types=[pltpu.VMEM((nh * SL,), jnp.float32),
                       pltpu.VMEM((SL,), jnp.float32)])
    def k(denp_hbm, den_hbm, acc_t, buf_t):
        wid = _wid()
        nbase = pl.multiple_of(wid * SL, SL)
        zv = jnp.zeros((LANES,), jnp.float32)

        def zb(i, carry):
            acc_t[pl.ds(i * LANES, LANES)] = zv
            return carry
        lax.fori_loop(0, nh * SL // LANES, zb, 0)

        def tb(t, carry):
            for h in range(nh):
                pltpu.sync_copy(
                    denp_hbm.at[pl.ds(t * (nh * NPAD) + h * NPAD + nbase, SL)],
                    buf_t)

                def ab(i, carry2, h=h):
                    sl = pl.ds(h * SL + i * LANES, LANES)
                    slb = pl.ds(i * LANES, LANES)
                    acc_t[sl] = acc_t[sl] + buf_t[slb]
                    return carry2
                lax.fori_loop(0, SL // LANES, ab, 0)
            return carry
        lax.fori_loop(0, NW, tb, 0)
        for h in range(nh):
            pltpu.sync_copy(acc_t.at[pl.ds(h * SL, SL)],
                            den_hbm.at[pl.ds(h * NPAD + nbase, SL)])

    return k(denp)


def _sc_spmm(nh, nchunks, epad, srcp, dstp, ee, den, hc):
    cps = nchunks // NC
    ept = epad // NS
    nblk = ept // KB

    @functools.partial(
        pl.kernel,
        out_type=jax.ShapeDtypeStruct((nchunks * NPAD, 128), jnp.float32),
        mesh=_mesh(),
        compiler_params=pltpu.CompilerParams(needs_layout_passes=False),
        scratch_types=[
            pltpu.VMEM_SHARED((NPAD, 128), jnp.float32),
            pltpu.VMEM((KB,), jnp.int32),
            pltpu.VMEM((KB,), jnp.int32),
            pltpu.VMEM((KB,), jnp.int32),
            pltpu.VMEM((KB,), jnp.float32),
            pltpu.VMEM((KB, 128), jnp.float32),
            pltpu.VMEM((NPAD,), jnp.float32),
            pltpu.VMEM((KB, 128), jnp.float32),
            pltpu.VMEM((TSL,), jnp.float32),
            pltpu.SemaphoreType.DMA,
        ])
    def k(src_hbm, dst_hbm, ee_hbm, den_hbm, h_hbm, out_hbm,
          acc_sh, src_t, dst_t, idx_t, ee_t, msg_t, den_t, zbuf_t, inv_t, sem):
        sc = lax.axis_index("c")
        sub = lax.axis_index("s")
        zv = jnp.zeros((LANES,), jnp.float32)

        def zrow(i, carry):
            for r in range(8):
                zbuf_t[i, pl.ds(r * LANES, LANES)] = zv
            return carry
        lax.fori_loop(0, KB, zrow, 0)

        for cj in range(cps):
            c = sc * cps + cj
            head = (c * nh) // nchunks
            cbase = pl.multiple_of(c * NPAD, NPAD)
            pltpu.sync_copy(
                den_hbm.at[pl.ds(pl.multiple_of(head * NPAD, NPAD), NPAD)],
                den_t)
            for j in range(TSL // KB):
                pltpu.sync_copy(
                    zbuf_t, acc_sh.at[pl.ds(sub * TSL + j * KB, KB)])
            plsc.subcore_barrier()

            ebase = pl.multiple_of(sub * ept, KB)

            def blk(bi, carry):
                off = pl.multiple_of(ebase + bi * KB, KB)
                pltpu.sync_copy(src_hbm.at[pl.ds(off, KB)], src_t)
                pltpu.sync_copy(dst_hbm.at[pl.ds(off, KB)], dst_t)
                pltpu.sync_copy(
                    ee_hbm.at[pl.ds(head * epad + off, KB)], ee_t)

                def vi(i, carry2):
                    sl = pl.ds(i * LANES, LANES)
                    idx_t[sl] = src_t[sl] + cbase
                    return carry2
                lax.fori_loop(0, KB // LANES, vi, 0)
                pltpu.async_copy(h_hbm.at[idx_t], msg_t, sem).wait()

                def ei(iv, carry2):
                    av = ee_t[pl.ds(iv * LANES, LANES)]
                    for l in range(LANES):
                        a = av[l]
                        i = iv * LANES + l
                        for r in range(8):
                            sl = pl.ds(r * LANES, LANES)
                            msg_t[i, sl] = msg_t[i, sl] * a
                    return carry2
                lax.fori_loop(0, KB // LANES, ei, 0)
                pltpu.sync_copy(msg_t, acc_sh.at[dst_t], add=True)
                return carry
            lax.fori_loop(0, nblk, blk, 0)
            plsc.subcore_barrier()

            rbase = sub * TSL

            def iv(i, carry):
                sl = pl.ds(i * LANES, LANES)
                inv_t[sl] = 1.0 / (den_t[pl.ds(rbase + i * LANES, LANES)]
                                   + 1e-16)
                return carry
            lax.fori_loop(0, TSL // LANES, iv, 0)

            for j in range(TSL // KB):
                pltpu.sync_copy(acc_sh.at[pl.ds(rbase + j * KB, KB)], msg_t)

                def rr(iv, carry, j=j):
                    sv = inv_t[pl.ds(j * KB + iv * LANES, LANES)]
                    for l in range(LANES):
                        s = sv[l]
                        i = iv * LANES + l
                        for r in range(8):
                            sl = pl.ds(r * LANES, LANES)
                            msg_t[i, sl] = msg_t[i, sl] * s
                    return carry
                lax.fori_loop(0, KB // LANES, rr, 0)
                pltpu.sync_copy(
                    msg_t,
                    out_hbm.at[pl.ds(cbase + rbase + j * KB, KB)])
            plsc.subcore_barrier()

    return k(srcp, dstp, ee, den, hc)


def _sc_lookup(x2, u_idx, i_idx, n_idx):
    PW = B // NW

    out_t = jax.ShapeDtypeStruct((2 * B, 128), jnp.float32)

    @functools.partial(
        pl.kernel,
        out_type=[out_t, out_t, out_t],
        mesh=_mesh(),
        compiler_params=pltpu.CompilerParams(needs_layout_passes=False),
        scratch_types=[
            pltpu.VMEM((PW,), jnp.int32),
            pltpu.VMEM((PW,), jnp.int32),
            pltpu.VMEM((PW, 128), jnp.float32),
            pltpu.SemaphoreType.DMA,
        ])
    def k(x2_hbm, u_hbm, i_hbm, n_hbm, ou_hbm, oi_hbm, on_hbm,
          idx_t, idxc_t, buf_t, sem):
        wid = _wid()
        wbase = pl.multiple_of(wid * PW, PW)
        for a, (src_idx, dst, off) in enumerate(
                [(u_hbm, ou_hbm, 0), (i_hbm, oi_hbm, N_USERS),
                 (n_hbm, on_hbm, N_USERS)]):
            pltpu.sync_copy(src_idx.at[pl.ds(wbase, PW)], idx_t)
            for c in range(2):
                def ai(i, carry, off=off, c=c):
                    sl = pl.ds(i * LANES, LANES)
                    idxc_t[sl] = idx_t[sl] + (off + c * NPAD)
                    return carry
                lax.fori_loop(0, PW // LANES, ai, 0)
                pltpu.async_copy(x2_hbm.at[idxc_t], buf_t, sem).wait()
                pltpu.sync_copy(buf_t, dst.at[pl.ds(c * B + wbase, PW)])

    return k(x2, u_idx, i_idx, n_idx)


def kernel(user_indices, item_indices, bundle_indices, item_indices_negative,
           bundle_indices_negative, edge_index, emb, W1, a_src1, a_dst1, b1,
           W2, a_src2, a_dst2, b2):
    src, dst = edge_index[0], edge_index[1]
    e = src.shape[0]
    gran = NW * KA
    epad = ((e + gran - 1) // gran) * gran

    srcp = jnp.pad(src.astype(jnp.int32), (0, epad - e))
    pad_dst = N_NODES + (jnp.arange(epad - e, dtype=jnp.int32)
                         % (NPAD - N_NODES))
    dstp = jnp.concatenate([dst.astype(jnp.int32), pad_dst])
    embp = jnp.pad(emb, ((0, NPAD - emb.shape[0]), (0, 0)))

    eye1 = jnp.eye(HEADS1, dtype=jnp.float32)
    AmatT1 = jnp.concatenate(
        [(eye1[:, :, None] * a_src1[None, :, :]).reshape(HEADS1, HEADS1 * OUT1),
         (eye1[:, :, None] * a_dst1[None, :, :]).reshape(HEADS1, HEADS1 * OUT1)],
        axis=0)
    AmatT2 = jnp.concatenate(
        [a_src2, a_dst2, jnp.zeros((6, OUT2), jnp.float32)], axis=0)

    h1c, aux1 = _tc1(embp, W1, b1.reshape(1, -1), AmatT1)
    ee1, denp1 = _sc_edge_pass(HEADS1, epad, srcp, dstp, aux1.reshape(-1))
    den1 = _sc_den_reduce(HEADS1, denp1)
    x1 = _sc_spmm(HEADS1, 8, epad, srcp, dstp, ee1, den1,
                  h1c.reshape(8 * NPAD, 128))

    h2c, aux2 = _tc2(x1.reshape(8, NPAD, 128), W2, b2.reshape(1, -1), AmatT2)
    ee2, denp2 = _sc_edge_pass(HEADS2, epad, srcp, dstp, aux2.reshape(-1))
    den2 = _sc_den_reduce(HEADS2, denp2)
    x2 = _sc_spmm(HEADS2, 2, epad, srcp, dstp, ee2, den2,
                  h2c.reshape(2 * NPAD, 128))

    ou, oi, on = _sc_lookup(x2, user_indices.astype(jnp.int32),
                            item_indices.astype(jnp.int32),
                            item_indices_negative.astype(jnp.int32))
    user_embeds = jnp.concatenate([ou[:B], ou[B:]], axis=1)
    item_embeds = jnp.concatenate([oi[:B], oi[B:]], axis=1)
    item_embeds_neg = jnp.concatenate([on[:B], on[B:]], axis=1)
    return (user_embeds, item_embeds, item_embeds_neg)
